# baseline (speedup 1.0000x reference)
import jax
import jax.numpy as jnp
from jax.experimental import pallas as pl


B, N, M, K = 32, 2048, 512, 8
BOTTLENECK = 1024


def _mlp_kernel(gf_ref, w1_ref, b1_ref, g1_ref, be1_ref,
                w2_ref, b2_ref, g2_ref, be2_ref,
                w3_ref, b3_ref, g3_ref, be3_ref,
                w4_ref, b4_ref, y_ref):
    def bn_relu(y, g, be):
        mean = jnp.mean(y, axis=0, keepdims=True)
        var = jnp.mean((y - mean) * (y - mean), axis=0, keepdims=True)
        return jax.nn.relu((y - mean) * jax.lax.rsqrt(var + 1e-5) * g + be)

    y = jnp.dot(gf_ref[...], w1_ref[...], preferred_element_type=jnp.float32)
    y = bn_relu(y + b1_ref[...], g1_ref[...], be1_ref[...])
    y = jnp.dot(y, w2_ref[...], preferred_element_type=jnp.float32)
    y = bn_relu(y + b2_ref[...], g2_ref[...], be2_ref[...])
    y = jnp.dot(y, w3_ref[...], preferred_element_type=jnp.float32)
    y = bn_relu(y + b3_ref[...], g3_ref[...], be3_ref[...])
    y = jnp.dot(y, w4_ref[...], preferred_element_type=jnp.float32)
    y_ref[...] = y + b4_ref[...]


def _merge_sorted(a, b):
    n = len(a)
    s = a + b[::-1]
    d = n
    while d >= 1:
        for i0 in range(0, 2 * n, 2 * d):
            for i in range(i0, i0 + d):
                lo = jnp.minimum(s[i], s[i + d])
                hi = jnp.maximum(s[i], s[i + d])
                s[i], s[i + d] = lo, hi
        d //= 2
    return s


def _bitonic_sort8(s):
    s = list(s)
    d = 4
    while d >= 1:
        for i0 in range(0, 8, 2 * d):
            for i in range(i0, i0 + d):
                lo = jnp.minimum(s[i], s[i + d])
                hi = jnp.maximum(s[i], s[i + d])
                s[i], s[i + d] = lo, hi
        d //= 2
    return s


def _proj_kernel(q_ref, xt_ref, isig_ref, out_ref):
    inv_sigma = isig_ref[0, 0]
    for s in range(q_ref.shape[0]):
        out_ref[s] = _soft_proj(q_ref[s], xt_ref[s], inv_sigma)


def _soft_proj(q, pt, inv_sigma):
    qp2 = jnp.dot(-2.0 * q, pt, preferred_element_type=jnp.float32)
    q2 = jnp.sum(q * q, axis=1, keepdims=True)
    p2 = jnp.sum(pt * pt, axis=0, keepdims=True)
    d2sel = (q2 + qp2) + p2

    cw = N // 16
    c = [d2sel[:, j * cw:(j + 1) * cw] for j in range(16)]
    pairs = [_merge_sorted([c[2 * j]], [c[2 * j + 1]]) for j in range(8)]
    quads = [_merge_sorted(pairs[2 * j], pairs[2 * j + 1]) for j in range(4)]
    octs = [_merge_sorted(quads[0], quads[1]),
            _merge_sorted(quads[2], quads[3])]
    low = [jnp.minimum(octs[0][i], octs[1][7 - i]) for i in range(8)]
    cand = _bitonic_sort8(low)

    t = jnp.min(cand[0], axis=1, keepdims=True)
    for i in range(2, K + 1):
        mm = None
        for cd in cand[:i]:
            x = jnp.where(cd <= t, jnp.inf, cd)
            mm = x if mm is None else jnp.minimum(mm, x)
        t = jnp.min(mm, axis=1, keepdims=True)

    nc2 = inv_sigma * (-1.4426950408889634)
    w = jnp.where(d2sel <= t, jnp.exp2(d2sel * nc2), 0.0)

    px = pt[0:1, :]
    py = pt[1:2, :]
    pz = pt[2:3, :]
    wsum = jnp.sum(w, axis=1, keepdims=True)
    ox = jnp.sum(w * px, axis=1, keepdims=True)
    oy = jnp.sum(w * py, axis=1, keepdims=True)
    oz = jnp.sum(w * pz, axis=1, keepdims=True)
    return jnp.concatenate([ox, oy, oz], axis=1) / wsum


def _run_mlp(global_feat, W1t, b1, g1, be1, W2t, b2, g2, be2, W3t, b3, g3,
             be3, W4t, b4):
    return pl.pallas_call(
        _mlp_kernel,
        out_shape=jax.ShapeDtypeStruct((B, 3 * M), jnp.float32),
    )(global_feat, W1t, b1, g1, be1, W2t, b2, g2, be2, W3t, b3, g3, be3,
      W4t, b4)


def _run_proj(generated, xt, inv_sigma, nb, bps=4):
    return pl.pallas_call(
        _proj_kernel,
        grid=(nb // bps,),
        in_specs=[
            pl.BlockSpec((bps, M, 3), lambda b: (b, 0, 0)),
            pl.BlockSpec((bps, 3, N), lambda b: (b, 0, 0)),
            pl.BlockSpec((1, 1), lambda b: (0, 0)),
        ],
        out_specs=pl.BlockSpec((bps, M, 3), lambda b: (b, 0, 0)),
        out_shape=jax.ShapeDtypeStruct((nb, M, 3), jnp.float32),
    )(generated, xt, inv_sigma)


def kernel(x, global_feat, W1, b1, g1, be1, W2, b2, g2, be2, W3, b3, g3, be3,
           W4, b4, temperature):
    f32 = jnp.float32
    sigma = jnp.maximum(temperature * temperature, 0.01)
    inv_sigma = (1.0 / sigma).reshape(1, 1).astype(f32)
    mlp_args = (global_feat, W1.T, b1.reshape(1, -1), g1.reshape(1, -1),
                be1.reshape(1, -1), W2.T, b2.reshape(1, -1),
                g2.reshape(1, -1), be2.reshape(1, -1), W3.T,
                b3.reshape(1, -1), g3.reshape(1, -1), be3.reshape(1, -1),
                W4.T, b4.reshape(1, -1))

    y = _run_mlp(*mlp_args)
    generated = jnp.transpose(y.reshape(B, 3, M), (0, 2, 1))
    proj = _run_proj(generated, jnp.transpose(x, (0, 2, 1)), inv_sigma, B)
    return generated, proj


# speedup vs baseline: 1.0580x; 1.0580x over previous
import jax
import jax.numpy as jnp
from jax.experimental import pallas as pl


B, N, M, K = 32, 2048, 512, 8
BOTTLENECK = 1024


def _mlp_kernel(gf_ref, w1_ref, b1_ref, g1_ref, be1_ref,
                w2_ref, b2_ref, g2_ref, be2_ref,
                w3_ref, b3_ref, g3_ref, be3_ref,
                w4_ref, b4_ref, y_ref):
    def bn_relu(y, g, be):
        mean = jnp.mean(y, axis=0, keepdims=True)
        var = jnp.mean((y - mean) * (y - mean), axis=0, keepdims=True)
        return jax.nn.relu((y - mean) * jax.lax.rsqrt(var + 1e-5) * g + be)

    y = jnp.dot(gf_ref[...], w1_ref[...], preferred_element_type=jnp.float32)
    y = bn_relu(y + b1_ref[...], g1_ref[...], be1_ref[...])
    y = jnp.dot(y, w2_ref[...], preferred_element_type=jnp.float32)
    y = bn_relu(y + b2_ref[...], g2_ref[...], be2_ref[...])
    y = jnp.dot(y, w3_ref[...], preferred_element_type=jnp.float32)
    y = bn_relu(y + b3_ref[...], g3_ref[...], be3_ref[...])
    y = jnp.dot(y, w4_ref[...], preferred_element_type=jnp.float32)
    y_ref[...] = y + b4_ref[...]


def _ce(x, y):
    return jnp.minimum(x, y), jnp.maximum(x, y)


def _merge22(a, b):
    l0, h0 = _ce(a[0], b[0])
    l1, h1 = _ce(a[1], b[1])
    m0, m1 = _ce(h0, l1)
    return [l0, m0, m1, h1]


def _merge44(a, b):
    e = _merge22([a[0], a[2]], [b[0], b[2]])
    o = _merge22([a[1], a[3]], [b[1], b[3]])
    c1, c2 = _ce(e[1], o[0])
    c3, c4 = _ce(e[2], o[1])
    c5, c6 = _ce(e[3], o[2])
    return [e[0], c1, c2, c3, c4, c5, c6, o[3]]


def _bitonic_sort8(s):
    s = list(s)
    d = 4
    while d >= 1:
        for i0 in range(0, 8, 2 * d):
            for i in range(i0, i0 + d):
                lo = jnp.minimum(s[i], s[i + d])
                hi = jnp.maximum(s[i], s[i + d])
                s[i], s[i + d] = lo, hi
        d //= 2
    return s


def _proj_kernel(q_ref, xt_ref, isig_ref, out_ref):
    inv_sigma = isig_ref[0, 0]
    for s in range(q_ref.shape[0]):
        out_ref[s] = _soft_proj(q_ref[s], xt_ref[s], inv_sigma)


def _soft_proj(q, pt, inv_sigma):
    qp2 = jnp.dot(-2.0 * q, pt, preferred_element_type=jnp.float32)
    q2 = jnp.sum(q * q, axis=1, keepdims=True)
    p2 = jnp.sum(pt * pt, axis=0, keepdims=True)
    d2sel = (q2 + qp2) + p2

    cw = N // 16
    c = [d2sel[:, j * cw:(j + 1) * cw] for j in range(16)]
    pairs = [list(_ce(c[2 * j], c[2 * j + 1])) for j in range(8)]
    quads = [_merge22(pairs[2 * j], pairs[2 * j + 1]) for j in range(4)]
    octs = [_merge44(quads[0], quads[1]), _merge44(quads[2], quads[3])]
    low = [jnp.minimum(octs[0][i], octs[1][7 - i]) for i in range(8)]
    cand = _bitonic_sort8(low)

    t = jnp.min(cand[0], axis=1, keepdims=True)
    for i in range(2, K + 1):
        mm = cand[i - 1]
        for cd in cand[:i - 1]:
            x = jnp.where(cd <= t, jnp.inf, cd)
            mm = jnp.minimum(mm, x)
        t = jnp.min(mm, axis=1, keepdims=True)

    nc2 = inv_sigma * (-1.4426950408889634)
    w = jnp.where(d2sel <= t, jnp.exp2(d2sel * nc2), 0.0)

    px = pt[0:1, :]
    py = pt[1:2, :]
    pz = pt[2:3, :]
    wsum = jnp.sum(w, axis=1, keepdims=True)
    ox = jnp.sum(w * px, axis=1, keepdims=True)
    oy = jnp.sum(w * py, axis=1, keepdims=True)
    oz = jnp.sum(w * pz, axis=1, keepdims=True)
    return jnp.concatenate([ox, oy, oz], axis=1) / wsum


def _run_mlp(global_feat, W1t, b1, g1, be1, W2t, b2, g2, be2, W3t, b3, g3,
             be3, W4t, b4):
    return pl.pallas_call(
        _mlp_kernel,
        out_shape=jax.ShapeDtypeStruct((B, 3 * M), jnp.float32),
    )(global_feat, W1t, b1, g1, be1, W2t, b2, g2, be2, W3t, b3, g3, be3,
      W4t, b4)


def _run_proj(generated, xt, inv_sigma, nb, bps=2):
    return pl.pallas_call(
        _proj_kernel,
        grid=(nb // bps,),
        in_specs=[
            pl.BlockSpec((bps, M, 3), lambda b: (b, 0, 0)),
            pl.BlockSpec((bps, 3, N), lambda b: (b, 0, 0)),
            pl.BlockSpec((1, 1), lambda b: (0, 0)),
        ],
        out_specs=pl.BlockSpec((bps, M, 3), lambda b: (b, 0, 0)),
        out_shape=jax.ShapeDtypeStruct((nb, M, 3), jnp.float32),
    )(generated, xt, inv_sigma)


def kernel(x, global_feat, W1, b1, g1, be1, W2, b2, g2, be2, W3, b3, g3, be3,
           W4, b4, temperature):
    f32 = jnp.float32
    sigma = jnp.maximum(temperature * temperature, 0.01)
    inv_sigma = (1.0 / sigma).reshape(1, 1).astype(f32)
    mlp_args = (global_feat, W1.T, b1.reshape(1, -1), g1.reshape(1, -1),
                be1.reshape(1, -1), W2.T, b2.reshape(1, -1),
                g2.reshape(1, -1), be2.reshape(1, -1), W3.T,
                b3.reshape(1, -1), g3.reshape(1, -1), be3.reshape(1, -1),
                W4.T, b4.reshape(1, -1))

    y = _run_mlp(*mlp_args)
    generated = jnp.transpose(y.reshape(B, 3, M), (0, 2, 1))
    proj = _run_proj(generated, jnp.transpose(x, (0, 2, 1)), inv_sigma, B)
    return generated, proj


# speedup vs baseline: 1.0623x; 1.0041x over previous
import jax
import jax.numpy as jnp
from jax.experimental import pallas as pl


B, N, M, K = 32, 2048, 512, 8
BOTTLENECK = 1024


def _mlp_kernel(gf_ref, w1_ref, b1_ref, g1_ref, be1_ref,
                w2_ref, b2_ref, g2_ref, be2_ref,
                w3_ref, b3_ref, g3_ref, be3_ref,
                w4_ref, b4_ref, y_ref):
    def bn_relu(y, g, be):
        mean = jnp.mean(y, axis=0, keepdims=True)
        var = jnp.mean((y - mean) * (y - mean), axis=0, keepdims=True)
        return jax.nn.relu((y - mean) * jax.lax.rsqrt(var + 1e-5) * g + be)

    y = jnp.dot(gf_ref[...], w1_ref[...], preferred_element_type=jnp.float32)
    y = bn_relu(y + b1_ref[...], g1_ref[...], be1_ref[...])
    y = jnp.dot(y, w2_ref[...], preferred_element_type=jnp.float32)
    y = bn_relu(y + b2_ref[...], g2_ref[...], be2_ref[...])
    y = jnp.dot(y, w3_ref[...], preferred_element_type=jnp.float32)
    y = bn_relu(y + b3_ref[...], g3_ref[...], be3_ref[...])
    y = jnp.dot(y, w4_ref[...], preferred_element_type=jnp.float32)
    y_ref[...] = y + b4_ref[...]


def _ce(x, y):
    return jnp.minimum(x, y), jnp.maximum(x, y)


def _merge22(a, b):
    l0, h0 = _ce(a[0], b[0])
    l1, h1 = _ce(a[1], b[1])
    m0, m1 = _ce(h0, l1)
    return [l0, m0, m1, h1]


def _merge44(a, b):
    e = _merge22([a[0], a[2]], [b[0], b[2]])
    o = _merge22([a[1], a[3]], [b[1], b[3]])
    c1, c2 = _ce(e[1], o[0])
    c3, c4 = _ce(e[2], o[1])
    c5, c6 = _ce(e[3], o[2])
    return [e[0], c1, c2, c3, c4, c5, c6, o[3]]


def _bitonic_sort8(s):
    s = list(s)
    d = 4
    while d >= 1:
        for i0 in range(0, 8, 2 * d):
            for i in range(i0, i0 + d):
                lo = jnp.minimum(s[i], s[i + d])
                hi = jnp.maximum(s[i], s[i + d])
                s[i], s[i + d] = lo, hi
        d //= 2
    return s


def _proj_kernel(q_ref, xt_ref, isig_ref, out_ref):
    inv_sigma = isig_ref[0, 0]
    for s in range(q_ref.shape[0]):
        out_ref[s] = _soft_proj(q_ref[s], xt_ref[s], inv_sigma)


def _soft_proj(q, pt, inv_sigma):
    qp2 = jnp.dot(-2.0 * q, pt, preferred_element_type=jnp.float32)
    q2 = jnp.sum(q * q, axis=1, keepdims=True)
    p2 = jnp.sum(pt * pt, axis=0, keepdims=True)
    d2sel = (q2 + qp2) + p2

    cw = N // 16
    c = [d2sel[:, j * cw:(j + 1) * cw] for j in range(16)]
    pairs = [list(_ce(c[2 * j], c[2 * j + 1])) for j in range(8)]
    quads = [_merge22(pairs[2 * j], pairs[2 * j + 1]) for j in range(4)]
    octs = [_merge44(quads[0], quads[1]), _merge44(quads[2], quads[3])]
    low = [jnp.minimum(octs[0][i], octs[1][7 - i]) for i in range(8)]
    cand = _bitonic_sort8(low)

    t = jnp.min(cand[0], axis=1, keepdims=True)
    for i in range(2, K + 1):
        mm = cand[i - 1]
        for cd in cand[:i - 1]:
            x = jnp.where(cd <= t, jnp.inf, cd)
            mm = jnp.minimum(mm, x)
        t = jnp.min(mm, axis=1, keepdims=True)

    nc2 = inv_sigma * (-1.4426950408889634)
    aw = ax = ay = az = None
    for j in range(16):
        sl = slice(j * cw, (j + 1) * cw)
        d2c = d2sel[:, sl]
        wc = jnp.where(d2c <= t, jnp.exp2(d2c * nc2), 0.0)
        wx = wc * pt[0:1, sl]
        wy = wc * pt[1:2, sl]
        wz = wc * pt[2:3, sl]
        if aw is None:
            aw, ax, ay, az = wc, wx, wy, wz
        else:
            aw += wc
            ax += wx
            ay += wy
            az += wz
    wsum = jnp.sum(aw, axis=1, keepdims=True)
    ox = jnp.sum(ax, axis=1, keepdims=True)
    oy = jnp.sum(ay, axis=1, keepdims=True)
    oz = jnp.sum(az, axis=1, keepdims=True)
    return jnp.concatenate([ox, oy, oz], axis=1) / wsum


def _run_mlp(global_feat, W1t, b1, g1, be1, W2t, b2, g2, be2, W3t, b3, g3,
             be3, W4t, b4):
    return pl.pallas_call(
        _mlp_kernel,
        out_shape=jax.ShapeDtypeStruct((B, 3 * M), jnp.float32),
    )(global_feat, W1t, b1, g1, be1, W2t, b2, g2, be2, W3t, b3, g3, be3,
      W4t, b4)


def _run_proj(generated, xt, inv_sigma, nb, bps=2):
    return pl.pallas_call(
        _proj_kernel,
        grid=(nb // bps,),
        in_specs=[
            pl.BlockSpec((bps, M, 3), lambda b: (b, 0, 0)),
            pl.BlockSpec((bps, 3, N), lambda b: (b, 0, 0)),
            pl.BlockSpec((1, 1), lambda b: (0, 0)),
        ],
        out_specs=pl.BlockSpec((bps, M, 3), lambda b: (b, 0, 0)),
        out_shape=jax.ShapeDtypeStruct((nb, M, 3), jnp.float32),
    )(generated, xt, inv_sigma)


def kernel(x, global_feat, W1, b1, g1, be1, W2, b2, g2, be2, W3, b3, g3, be3,
           W4, b4, temperature):
    f32 = jnp.float32
    sigma = jnp.maximum(temperature * temperature, 0.01)
    inv_sigma = (1.0 / sigma).reshape(1, 1).astype(f32)
    mlp_args = (global_feat, W1.T, b1.reshape(1, -1), g1.reshape(1, -1),
                be1.reshape(1, -1), W2.T, b2.reshape(1, -1),
                g2.reshape(1, -1), be2.reshape(1, -1), W3.T,
                b3.reshape(1, -1), g3.reshape(1, -1), be3.reshape(1, -1),
                W4.T, b4.reshape(1, -1))

    y = _run_mlp(*mlp_args)
    generated = jnp.transpose(y.reshape(B, 3, M), (0, 2, 1))
    proj = _run_proj(generated, jnp.transpose(x, (0, 2, 1)), inv_sigma, B)
    return generated, proj
